# baseline (speedup 1.0000x reference)
import functools

import jax
import jax.numpy as jnp
from jax.experimental import pallas as pl
from jax.experimental.pallas import tpu as pltpu

C_IN = 32
C_OUT = 32
MIN_V = 0.1
MAX_V = 0.8


def _conv_pipe_kernel(x_hbm, s_ref, w_ref, b_ref, o_hbm,
                      x_buf, o_buf, in_sem, out_sem, *, ho, wo, n_steps):
    half = pl.program_id(0)
    base = half * n_steps

    def dma_in(slot, step):
        pltpu.make_async_copy(
            x_hbm.at[base + step], x_buf.at[slot], in_sem.at[slot]).start()

    def wait_in(slot):
        pltpu.make_async_copy(
            x_hbm.at[base], x_buf.at[slot], in_sem.at[slot]).wait()

    def dma_out(slot, step):
        pltpu.make_async_copy(
            o_buf.at[slot], o_hbm.at[base + step], out_sem.at[slot]).start()

    def wait_out(slot):
        pltpu.make_async_copy(
            o_buf.at[slot], o_hbm.at[base], out_sem.at[slot]).wait()

    def compute(slot):
        xr = x_buf[slot, :, 1 : 2 * ho : 2, :]
        a2 = xr.reshape(C_IN * ho, 2 * wo)
        p = jnp.dot(a2, s_ref[...], preferred_element_type=jnp.float32)
        p3 = p.reshape(C_IN, ho, 2 * wo)
        tl = p3[:, :, :wo].reshape(C_IN, ho * wo)
        tr = p3[:, :, wo:].reshape(C_IN, ho * wo)
        zc = jnp.zeros((C_IN, wo), jnp.float32)
        tlu = jnp.concatenate([zc, tl[:, :-wo]], axis=1)
        tru = jnp.concatenate([zc, tr[:, :-wo]], axis=1)
        acc = jnp.dot(w_ref[0], tlu, preferred_element_type=jnp.float32)
        acc = acc + jnp.dot(w_ref[1], tru, preferred_element_type=jnp.float32)
        acc = acc + jnp.dot(w_ref[2], tl, preferred_element_type=jnp.float32)
        acc = acc + jnp.dot(w_ref[3], tr, preferred_element_type=jnp.float32)
        acc = jnp.clip(acc + b_ref[...], MIN_V, MAX_V)
        o_buf[slot] = acc.reshape(C_OUT, ho, wo)

    dma_in(0, 0)

    def body(step, _):
        cur = jax.lax.rem(step, 2)
        nxt = jax.lax.rem(step + 1, 2)

        @pl.when(step + 1 < n_steps)
        def _():
            dma_in(nxt, step + 1)

        wait_in(cur)

        @pl.when(step >= 2)
        def _():
            wait_out(cur)

        compute(cur)
        dma_out(cur, step)
        return ()

    jax.lax.fori_loop(0, n_steps, body, ())
    wait_out(jax.lax.rem(n_steps - 2, 2))
    wait_out(jax.lax.rem(n_steps - 1, 2))


def kernel(x, weight, bias):
    n, _, h, w = x.shape
    ho, wo = h // 2, w // 2

    wt = jnp.transpose(weight, (2, 3, 0, 1)).reshape(4, C_OUT, C_IN)
    b2 = bias.reshape(C_OUT, 1).astype(jnp.float32)
    rows = jnp.arange(w)[:, None]
    cols = jnp.arange(2 * wo)[None, :]
    sel = (rows == (2 * (cols % wo) - 1 + 2 * (cols // wo))).astype(jnp.float32)

    return pl.pallas_call(
        functools.partial(_conv_pipe_kernel, ho=ho, wo=wo, n_steps=n // 2),
        out_shape=jax.ShapeDtypeStruct((n, C_OUT, ho, wo), jnp.float32),
        grid=(2,),
        in_specs=[
            pl.BlockSpec(memory_space=pltpu.MemorySpace.HBM),
            pl.BlockSpec((w, 2 * wo), lambda i: (0, 0)),
            pl.BlockSpec((4, C_OUT, C_IN), lambda i: (0, 0, 0)),
            pl.BlockSpec((C_OUT, 1), lambda i: (0, 0)),
        ],
        out_specs=pl.BlockSpec(memory_space=pltpu.MemorySpace.HBM),
        scratch_shapes=[
            pltpu.VMEM((2, C_IN, 2 * ho, 2 * wo), jnp.float32),
            pltpu.VMEM((2, C_OUT, ho, wo), jnp.float32),
            pltpu.SemaphoreType.DMA((2,)),
            pltpu.SemaphoreType.DMA((2,)),
        ],
        compiler_params=pltpu.CompilerParams(
            dimension_semantics=("parallel",)),
    )(x, sel, wt, b2)


# speedup vs baseline: 1.0985x; 1.0985x over previous
import functools

import jax
import jax.numpy as jnp
from jax.experimental import pallas as pl
from jax.experimental.pallas import tpu as pltpu

C_IN = 32
C_OUT = 32
MIN_V = 0.1
MAX_V = 0.8


def _conv_pipe_kernel(x_hbm, s_ref, w_ref, b_ref, o_hbm,
                      x_buf, o_buf, in_sem, out_sem, *, ho, wo, n_steps):
    half = pl.program_id(0)
    base = half * n_steps

    xp = x_hbm.reshape(2 * n_steps, C_IN, ho, 2, 2 * wo)

    def dma_in(slot, step):
        pltpu.make_async_copy(
            xp.at[base + step, :, :, 1, :], x_buf.at[slot],
            in_sem.at[slot]).start()

    def wait_in(slot):
        pltpu.make_async_copy(
            xp.at[base, :, :, 1, :], x_buf.at[slot], in_sem.at[slot]).wait()

    def dma_out(slot, step):
        pltpu.make_async_copy(
            o_buf.at[slot], o_hbm.at[base + step], out_sem.at[slot]).start()

    def wait_out(slot):
        pltpu.make_async_copy(
            o_buf.at[slot], o_hbm.at[base], out_sem.at[slot]).wait()

    def compute(slot):
        xr = x_buf[slot]
        a2 = xr.reshape(C_IN * ho, 2 * wo)
        p = jnp.dot(a2, s_ref[...], preferred_element_type=jnp.float32)
        p3 = p.reshape(C_IN, ho, 2 * wo)
        tl = p3[:, :, :wo].reshape(C_IN, ho * wo)
        tr = p3[:, :, wo:].reshape(C_IN, ho * wo)
        zc = jnp.zeros((C_IN, wo), jnp.float32)
        tlu = jnp.concatenate([zc, tl[:, :-wo]], axis=1)
        tru = jnp.concatenate([zc, tr[:, :-wo]], axis=1)
        acc = jnp.dot(w_ref[0], tlu, preferred_element_type=jnp.float32)
        acc = acc + jnp.dot(w_ref[1], tru, preferred_element_type=jnp.float32)
        acc = acc + jnp.dot(w_ref[2], tl, preferred_element_type=jnp.float32)
        acc = acc + jnp.dot(w_ref[3], tr, preferred_element_type=jnp.float32)
        acc = jnp.clip(acc + b_ref[...], MIN_V, MAX_V)
        o_buf[slot] = acc.reshape(C_OUT, ho, wo)

    dma_in(0, 0)

    def body(step, _):
        cur = jax.lax.rem(step, 2)
        nxt = jax.lax.rem(step + 1, 2)

        @pl.when(step + 1 < n_steps)
        def _():
            dma_in(nxt, step + 1)

        wait_in(cur)

        @pl.when(step >= 2)
        def _():
            wait_out(cur)

        compute(cur)
        dma_out(cur, step)
        return ()

    jax.lax.fori_loop(0, n_steps, body, ())
    wait_out(jax.lax.rem(n_steps - 2, 2))
    wait_out(jax.lax.rem(n_steps - 1, 2))


def kernel(x, weight, bias):
    n, _, h, w = x.shape
    ho, wo = h // 2, w // 2

    wt = jnp.transpose(weight, (2, 3, 0, 1)).reshape(4, C_OUT, C_IN)
    b2 = bias.reshape(C_OUT, 1).astype(jnp.float32)
    rows = jnp.arange(w)[:, None]
    cols = jnp.arange(2 * wo)[None, :]
    sel = (rows == (2 * (cols % wo) - 1 + 2 * (cols // wo))).astype(jnp.float32)

    return pl.pallas_call(
        functools.partial(_conv_pipe_kernel, ho=ho, wo=wo, n_steps=n // 2),
        out_shape=jax.ShapeDtypeStruct((n, C_OUT, ho, wo), jnp.float32),
        grid=(2,),
        in_specs=[
            pl.BlockSpec(memory_space=pltpu.MemorySpace.HBM),
            pl.BlockSpec((w, 2 * wo), lambda i: (0, 0)),
            pl.BlockSpec((4, C_OUT, C_IN), lambda i: (0, 0, 0)),
            pl.BlockSpec((C_OUT, 1), lambda i: (0, 0)),
        ],
        out_specs=pl.BlockSpec(memory_space=pltpu.MemorySpace.HBM),
        scratch_shapes=[
            pltpu.VMEM((2, C_IN, ho, 2 * wo), jnp.float32),
            pltpu.VMEM((2, C_OUT, ho, wo), jnp.float32),
            pltpu.SemaphoreType.DMA((2,)),
            pltpu.SemaphoreType.DMA((2,)),
        ],
        compiler_params=pltpu.CompilerParams(
            dimension_semantics=("parallel",)),
    )(x, sel, wt, b2)
